# baseline (speedup 1.0000x reference)
import functools

import jax
import jax.numpy as jnp
from jax import lax
from jax.experimental import pallas as pl
from jax.experimental.pallas import tpu as pltpu
from jax.experimental.pallas import tpu_sc as plsc

B = 1048576
NC = 2
NS = 16
NW = NC * NS
ROWS_PER_W = B // NW
CHUNK = 2048
NCHUNK = ROWS_PER_W // CHUNK
GROUPS = CHUNK // 16


def _round_bf16(v):
    u = plsc.bitcast(v, jnp.uint32)
    u = u + (jnp.uint32(0x7FFF) + ((u >> jnp.uint32(16)) & jnp.uint32(1)))
    u = u & jnp.uint32(0xFFFF0000)
    return plsc.bitcast(u, jnp.float32)


def _sc_body(x_hbm, wb_hbm, val_hbm, idx_hbm, xbuf, vbuf, ibuf, wv):
    wid = lax.axis_index("s") * NC + lax.axis_index("c")
    pltpu.sync_copy(wb_hbm, wv)

    lane = lax.iota(jnp.int32, 16)
    lane20 = lane * 20
    lane3 = lane * 3

    Pv = [wv[t, :] for t in range(20)]
    Wv = [_round_bf16(wv[20 + t, :]) for t in range(20)]
    Bv = [wv[40 + m, :] for m in range(5)]

    def do_chunk(c, carry):
        row0 = wid * ROWS_PER_W + c * CHUNK
        pltpu.sync_copy(x_hbm.at[pl.ds(row0 * 20, CHUNK * 20)], xbuf)

        def do_group(g, carry2):
            gbase = g * (16 * 20)
            xs = [plsc.load_gather(xbuf, [lane20 + (gbase + t)])
                  for t in range(20)]
            xp = [_round_bf16(xs[t] + Pv[t]) for t in range(20)]
            s = []
            for j in range(5):
                sj = None
                for m in range(5):
                    acc = Bv[m]
                    for k in range(4):
                        acc = acc + xp[4 * j + k] * Wv[4 * m + k]
                    r = jnp.maximum(acc, 0.0)
                    sj = r if sj is None else sj + r
                s.append(sj)
            one = jnp.full((16,), 1, jnp.int32)
            zero = jnp.full((16,), 0, jnp.int32)
            R = [zero] * 5
            for j in range(5):
                for l in range(j + 1, 5):
                    cge = s[j] >= s[l]
                    R[l] = R[l] + jnp.where(cge, one, zero)
                    R[j] = R[j] + jnp.where(cge, zero, one)
            obase = lane3 + (g * (16 * 3))
            for p in range(3):
                pv = jnp.full((16,), p, jnp.int32)
                v = s[4]
                i = jnp.full((16,), 4, jnp.int32)
                for j in (3, 2, 1, 0):
                    cj = R[j] == pv
                    jv = jnp.full((16,), j, jnp.int32)
                    v = jnp.where(cj, s[j], v)
                    i = jnp.where(cj, jv, i)
                plsc.store_scatter(vbuf, [obase + p], v)
                plsc.store_scatter(ibuf, [obase + p], i)
            return carry2

        lax.fori_loop(0, GROUPS, do_group, 0)
        pltpu.sync_copy(vbuf, val_hbm.at[pl.ds(row0 * 3, CHUNK * 3)])
        pltpu.sync_copy(ibuf, idx_hbm.at[pl.ds(row0 * 3, CHUNK * 3)])
        return carry

    lax.fori_loop(0, NCHUNK, do_chunk, 0)


_sc_call = pl.kernel(
    _sc_body,
    out_type=(
        jax.ShapeDtypeStruct((B * 3,), jnp.float32),
        jax.ShapeDtypeStruct((B * 3,), jnp.int32),
    ),
    mesh=plsc.VectorSubcoreMesh(
        core_axis_name="c", subcore_axis_name="s",
        num_cores=NC, num_subcores=NS,
    ),
    scratch_types=[
        pltpu.VMEM((CHUNK * 20,), jnp.float32),
        pltpu.VMEM((CHUNK * 3,), jnp.float32),
        pltpu.VMEM((CHUNK * 3,), jnp.int32),
        pltpu.VMEM((45, 16), jnp.float32),
    ],
    compiler_params=pltpu.CompilerParams(needs_layout_passes=False),
)


@jax.jit
def kernel(x, param, W, b):
    xf = x.reshape(B * 20)
    wb = jnp.concatenate([
        param.T.reshape(20), W.reshape(20), b.reshape(5)]).astype(jnp.float32)
    wb16 = jnp.broadcast_to(wb[:, None], (45, 16))
    vals, idxs = _sc_call(xf, wb16)
    return vals.reshape(B, 3), idxs.reshape(B, 3)
